# Initial kernel scaffold; baseline (speedup 1.0000x reference)
import jax
import jax.numpy as jnp
from jax.experimental import pallas as pl


def kernel(x, edge_index, W1, b1, W2, b2):
    raise NotImplementedError("write your pallas kernel here")


# speedup vs baseline: 19.1172x; 19.1172x over previous
import functools

import jax
import jax.numpy as jnp
from jax import lax
from jax.experimental import pallas as pl
from jax.experimental.pallas import tpu as pltpu
from jax.experimental.pallas import tpu_sc as plsc

_NC = 2
_NS = 16
_NW = _NC * _NS
_W = 80


def _make_edge_agg(Np, D, rows_per_worker, rows_per_tile):
  mesh = plsc.VectorSubcoreMesh(core_axis_name="c", subcore_axis_name="s")

  @functools.partial(
      pl.kernel,
      out_type=jax.ShapeDtypeStruct((_NC, Np, D), jnp.float32),
      mesh=mesh,
      scratch_types=[
          pltpu.VMEM((rows_per_worker, _W), jnp.int32),
          pltpu.VMEM((rows_per_worker, _W), jnp.int32),
          pltpu.VMEM((_W, D), jnp.float32),
          pltpu.VMEM_SHARED((Np, D), jnp.float32),
      ],
  )
  def agg(rows_hbm, srcb, dstb, zeros_hbm, out, src_v, dst_v, buf_v, accum):
    c = lax.axis_index("c")
    s = lax.axis_index("s")
    wid = c * _NS + s
    pltpu.sync_copy(zeros_hbm.at[pl.ds(s * rows_per_tile, rows_per_tile)],
                    accum.at[pl.ds(s * rows_per_tile, rows_per_tile)])
    pltpu.sync_copy(srcb.at[wid], src_v)
    pltpu.sync_copy(dstb.at[wid], dst_v)
    plsc.subcore_barrier()

    def win(j, carry):
      pltpu.sync_copy(rows_hbm.at[src_v.at[j]], buf_v)
      pltpu.sync_copy(buf_v, accum.at[dst_v.at[j]], add=True)
      return carry

    lax.fori_loop(0, rows_per_worker, win, 0)
    plsc.subcore_barrier()
    pltpu.sync_copy(accum.at[pl.ds(s * rows_per_tile, rows_per_tile)],
                    out.at[c, pl.ds(s * rows_per_tile, rows_per_tile)])

  return agg


def _make_deg(Np, rows_per_worker, rows_per_tile):
  mesh = plsc.VectorSubcoreMesh(core_axis_name="c", subcore_axis_name="s")

  @functools.partial(
      pl.kernel,
      out_type=jax.ShapeDtypeStruct((_NC * Np,), jnp.float32),
      mesh=mesh,
      scratch_types=[
          pltpu.VMEM((rows_per_worker, _W), jnp.int32),
          pltpu.VMEM((_W,), jnp.float32),
          pltpu.VMEM((rows_per_tile,), jnp.float32),
          pltpu.VMEM_SHARED((Np,), jnp.float32),
      ],
  )
  def deg(dstb, out, dst_v, ones_v, zbuf, accum):
    c = lax.axis_index("c")
    s = lax.axis_index("s")
    wid = c * _NS + s
    ones16 = jnp.ones((16,), jnp.float32)
    zero16 = jnp.zeros((16,), jnp.float32)

    def fill_ones(i, carry):
      ones_v[pl.ds(i * 16, 16)] = ones16
      return carry

    lax.fori_loop(0, _W // 16, fill_ones, 0)

    def fill_zero(i, carry):
      zbuf[pl.ds(i * 16, 16)] = zero16
      return carry

    lax.fori_loop(0, rows_per_tile // 16, fill_zero, 0)
    pltpu.sync_copy(zbuf, accum.at[pl.ds(s * rows_per_tile, rows_per_tile)])
    pltpu.sync_copy(dstb.at[wid], dst_v)
    plsc.subcore_barrier()

    def win(j, carry):
      pltpu.sync_copy(ones_v, accum.at[dst_v.at[j]], add=True)
      return carry

    lax.fori_loop(0, rows_per_worker, win, 0)
    plsc.subcore_barrier()
    obase = pl.multiple_of(c * Np + s * rows_per_tile, 128)
    pltpu.sync_copy(accum.at[pl.ds(s * rows_per_tile, rows_per_tile)],
                    out.at[pl.ds(obase, rows_per_tile)])

  return deg


def _k1_body(x_ref, w1_ref, dinv_ref, hs_ref):
  h = jnp.dot(x_ref[...], w1_ref[...], preferred_element_type=jnp.float32)
  hs_ref[...] = h * dinv_ref[...]


def _k2_body(a1_ref, hs_ref, dinv_ref, b1_ref, h2s_ref):
  dinv = dinv_ref[...]
  raw = a1_ref[0] + a1_ref[1] + hs_ref[...]
  h2 = jnp.maximum(raw * dinv + b1_ref[...], 0.0)
  h2s_ref[...] = h2 * dinv


def _k3_body(a2_ref, h2s_ref, dinv_ref, w2_ref, b2_ref, out_ref, logp_ref):
  raw = (a2_ref[0] + a2_ref[1] + h2s_ref[...]) * dinv_ref[...]
  o40 = jnp.dot(raw, w2_ref[...],
                preferred_element_type=jnp.float32) + b2_ref[...]
  m = jnp.max(o40, axis=1, keepdims=True)
  lse = jnp.log(jnp.sum(jnp.exp(o40 - m), axis=1, keepdims=True))
  out_ref[...] = o40
  logp_ref[...] = o40 - m - lse


def kernel(x, edge_index, W1, b1, W2, b2):
  N, F = x.shape
  H = W1.shape[1]
  C = W2.shape[1]
  E = edge_index.shape[1]
  Np = ((N + _NS * 128 - 1) // (_NS * 128)) * (_NS * 128)

  n_rows = E // _W
  rows_per_worker = n_rows // _NW
  rows_per_tile = Np // _NS

  xp = jnp.pad(x, ((0, Np - N), (0, 0)))
  src3 = edge_index[0].reshape(_NW, rows_per_worker, _W)
  dst3 = edge_index[1].reshape(_NW, rows_per_worker, _W)
  zH = jnp.zeros((Np, H), jnp.float32)
  b1r = b1.reshape(1, H)
  b2r = b2.reshape(1, C)

  deg_fn = _make_deg(Np, rows_per_worker, rows_per_tile)
  aggH_fn = _make_edge_agg(Np, H, rows_per_worker, rows_per_tile)

  degp = deg_fn(dst3)
  dinv_col = lax.rsqrt(degp[:Np] + degp[Np:] + 1.0)[:, None]

  B = 1024
  grid = (Np // B,)
  hs = pl.pallas_call(
      _k1_body,
      grid=grid,
      in_specs=[
          pl.BlockSpec((B, F), lambda i: (i, 0)),
          pl.BlockSpec((F, H), lambda i: (0, 0)),
          pl.BlockSpec((B, 1), lambda i: (i, 0)),
      ],
      out_specs=pl.BlockSpec((B, H), lambda i: (i, 0)),
      out_shape=jax.ShapeDtypeStruct((Np, H), jnp.float32),
  )(xp, W1, dinv_col)

  a1 = aggH_fn(hs, src3, dst3, zH)

  h2s = pl.pallas_call(
      _k2_body,
      grid=grid,
      in_specs=[
          pl.BlockSpec((_NC, B, H), lambda i: (0, i, 0)),
          pl.BlockSpec((B, H), lambda i: (i, 0)),
          pl.BlockSpec((B, 1), lambda i: (i, 0)),
          pl.BlockSpec((1, H), lambda i: (0, 0)),
      ],
      out_specs=pl.BlockSpec((B, H), lambda i: (i, 0)),
      out_shape=jax.ShapeDtypeStruct((Np, H), jnp.float32),
  )(a1, hs, dinv_col, b1r)

  a2 = aggH_fn(h2s, src3, dst3, zH)

  out2, logp = pl.pallas_call(
      _k3_body,
      grid=grid,
      in_specs=[
          pl.BlockSpec((_NC, B, H), lambda i: (0, i, 0)),
          pl.BlockSpec((B, H), lambda i: (i, 0)),
          pl.BlockSpec((B, 1), lambda i: (i, 0)),
          pl.BlockSpec((H, C), lambda i: (0, 0)),
          pl.BlockSpec((1, C), lambda i: (0, 0)),
      ],
      out_specs=[
          pl.BlockSpec((B, C), lambda i: (i, 0)),
          pl.BlockSpec((B, C), lambda i: (i, 0)),
      ],
      out_shape=[
          jax.ShapeDtypeStruct((Np, C), jnp.float32),
          jax.ShapeDtypeStruct((Np, C), jnp.float32),
      ],
  )(a2, h2s, dinv_col, W2, b2r)

  return (out2[:N], logp[:N])


# speedup vs baseline: 28.4448x; 1.4879x over previous
import functools

import jax
import jax.numpy as jnp
from jax import lax
from jax.experimental import pallas as pl
from jax.experimental.pallas import tpu as pltpu
from jax.experimental.pallas import tpu_sc as plsc

_NC = 2
_NS = 16
_NW = _NC * _NS
_W = 80


def _make_edge_agg(Np, D, rows_per_worker, rows_per_tile):
  mesh = plsc.VectorSubcoreMesh(core_axis_name="c", subcore_axis_name="s")

  _IC = 64
  chunks = []
  base = 0
  while base < rows_per_worker:
    n = min(_IC, rows_per_worker - base)
    chunks.append((base, n))
    base += n

  @functools.partial(
      pl.kernel,
      out_type=jax.ShapeDtypeStruct((_NC, Np, D), jnp.float32),
      mesh=mesh,
      scratch_types=[
          pltpu.VMEM((_IC, _W), jnp.int32),
          pltpu.VMEM((_IC, _W), jnp.int32),
          pltpu.VMEM((_W, D), jnp.float32),
          pltpu.VMEM((_W, D), jnp.float32),
          pltpu.VMEM_SHARED((Np, D), jnp.float32),
          pltpu.SemaphoreType.DMA,
          pltpu.SemaphoreType.DMA,
      ],
  )
  def agg(rows_hbm, srcb, dstb, zeros_hbm, out,
          src_v, dst_v, buf0, buf1, accum, g0, g1):
    c = lax.axis_index("c")
    s = lax.axis_index("s")
    wid = c * _NS + s
    pltpu.sync_copy(zeros_hbm.at[pl.ds(s * rows_per_tile, rows_per_tile)],
                    accum.at[pl.ds(s * rows_per_tile, rows_per_tile)])
    plsc.subcore_barrier()

    for base, nw in chunks:
      pltpu.sync_copy(srcb.at[wid, pl.ds(base, nw)], src_v.at[pl.ds(0, nw)])
      pltpu.sync_copy(dstb.at[wid, pl.ds(base, nw)], dst_v.at[pl.ds(0, nw)])
      npair = (nw - 1) // 2
      pltpu.async_copy(rows_hbm.at[src_v.at[0]], buf0, g0)

      def pair(t, carry):
        j0 = 2 * t
        pltpu.async_copy(rows_hbm.at[src_v.at[j0 + 1]], buf1, g1)
        pltpu.make_async_copy(rows_hbm.at[src_v.at[j0]], buf0, g0).wait()
        pltpu.sync_copy(buf0, accum.at[dst_v.at[j0]], add=True)
        pltpu.async_copy(rows_hbm.at[src_v.at[j0 + 2]], buf0, g0)
        pltpu.make_async_copy(rows_hbm.at[src_v.at[j0 + 1]], buf1, g1).wait()
        pltpu.sync_copy(buf1, accum.at[dst_v.at[j0 + 1]], add=True)
        return carry

      lax.fori_loop(0, npair, pair, 0)
      j = 2 * npair
      pltpu.make_async_copy(rows_hbm.at[src_v.at[j]], buf0, g0).wait()
      if nw - j == 2:
        pltpu.async_copy(rows_hbm.at[src_v.at[j + 1]], buf1, g1)
        pltpu.sync_copy(buf0, accum.at[dst_v.at[j]], add=True)
        pltpu.make_async_copy(rows_hbm.at[src_v.at[j + 1]], buf1, g1).wait()
        pltpu.sync_copy(buf1, accum.at[dst_v.at[j + 1]], add=True)
      else:
        pltpu.sync_copy(buf0, accum.at[dst_v.at[j]], add=True)

    plsc.subcore_barrier()
    pltpu.sync_copy(accum.at[pl.ds(s * rows_per_tile, rows_per_tile)],
                    out.at[c, pl.ds(s * rows_per_tile, rows_per_tile)])

  return agg


def _make_deg(Np, rows_per_worker, rows_per_tile):
  mesh = plsc.VectorSubcoreMesh(core_axis_name="c", subcore_axis_name="s")

  @functools.partial(
      pl.kernel,
      out_type=jax.ShapeDtypeStruct((_NC * Np,), jnp.float32),
      mesh=mesh,
      scratch_types=[
          pltpu.VMEM((rows_per_worker, _W), jnp.int32),
          pltpu.VMEM((_W,), jnp.float32),
          pltpu.VMEM((rows_per_tile,), jnp.float32),
          pltpu.VMEM_SHARED((Np,), jnp.float32),
      ],
  )
  def deg(dstb, out, dst_v, ones_v, zbuf, accum):
    c = lax.axis_index("c")
    s = lax.axis_index("s")
    wid = c * _NS + s
    ones16 = jnp.ones((16,), jnp.float32)
    zero16 = jnp.zeros((16,), jnp.float32)

    def fill_ones(i, carry):
      ones_v[pl.ds(i * 16, 16)] = ones16
      return carry

    lax.fori_loop(0, _W // 16, fill_ones, 0)

    def fill_zero(i, carry):
      zbuf[pl.ds(i * 16, 16)] = zero16
      return carry

    lax.fori_loop(0, rows_per_tile // 16, fill_zero, 0)
    pltpu.sync_copy(zbuf, accum.at[pl.ds(s * rows_per_tile, rows_per_tile)])
    pltpu.sync_copy(dstb.at[wid], dst_v)
    plsc.subcore_barrier()

    def win(j, carry):
      pltpu.sync_copy(ones_v, accum.at[dst_v.at[j]], add=True)
      return carry

    lax.fori_loop(0, rows_per_worker, win, 0)
    plsc.subcore_barrier()
    obase = pl.multiple_of(c * Np + s * rows_per_tile, 128)
    pltpu.sync_copy(accum.at[pl.ds(s * rows_per_tile, rows_per_tile)],
                    out.at[pl.ds(obase, rows_per_tile)])

  return deg


def _k1_body(x_ref, w1_ref, dinv_ref, hs_ref):
  h = jnp.dot(x_ref[...], w1_ref[...], preferred_element_type=jnp.float32)
  hs_ref[...] = h * dinv_ref[...]


def _k2_body(a1_ref, hs_ref, dinv_ref, b1_ref, h2s_ref):
  dinv = dinv_ref[...]
  raw = a1_ref[0] + a1_ref[1] + hs_ref[...]
  h2 = jnp.maximum(raw * dinv + b1_ref[...], 0.0)
  h2s_ref[...] = h2 * dinv


def _k3_body(a2_ref, h2s_ref, dinv_ref, w2_ref, b2_ref, out_ref, logp_ref):
  raw = (a2_ref[0] + a2_ref[1] + h2s_ref[...]) * dinv_ref[...]
  o40 = jnp.dot(raw, w2_ref[...],
                preferred_element_type=jnp.float32) + b2_ref[...]
  m = jnp.max(o40, axis=1, keepdims=True)
  lse = jnp.log(jnp.sum(jnp.exp(o40 - m), axis=1, keepdims=True))
  out_ref[...] = o40
  logp_ref[...] = o40 - m - lse


def kernel(x, edge_index, W1, b1, W2, b2):
  N, F = x.shape
  H = W1.shape[1]
  C = W2.shape[1]
  E = edge_index.shape[1]
  Np = ((N + _NS * 128 - 1) // (_NS * 128)) * (_NS * 128)

  n_rows = E // _W
  rows_per_worker = n_rows // _NW
  rows_per_tile = Np // _NS

  xp = jnp.pad(x, ((0, Np - N), (0, 0)))
  src3 = edge_index[0].reshape(_NW, rows_per_worker, _W)
  dst3 = edge_index[1].reshape(_NW, rows_per_worker, _W)
  zH = jnp.zeros((Np, H), jnp.float32)
  b1r = b1.reshape(1, H)
  b2r = b2.reshape(1, C)

  deg_fn = _make_deg(Np, rows_per_worker, rows_per_tile)
  aggH_fn = _make_edge_agg(Np, H, rows_per_worker, rows_per_tile)

  degp = deg_fn(dst3)
  dinv_col = lax.rsqrt(degp[:Np] + degp[Np:] + 1.0)[:, None]

  B = 1024
  grid = (Np // B,)
  hs = pl.pallas_call(
      _k1_body,
      grid=grid,
      in_specs=[
          pl.BlockSpec((B, F), lambda i: (i, 0)),
          pl.BlockSpec((F, H), lambda i: (0, 0)),
          pl.BlockSpec((B, 1), lambda i: (i, 0)),
      ],
      out_specs=pl.BlockSpec((B, H), lambda i: (i, 0)),
      out_shape=jax.ShapeDtypeStruct((Np, H), jnp.float32),
  )(xp, W1, dinv_col)

  a1 = aggH_fn(hs, src3, dst3, zH)

  h2s = pl.pallas_call(
      _k2_body,
      grid=grid,
      in_specs=[
          pl.BlockSpec((_NC, B, H), lambda i: (0, i, 0)),
          pl.BlockSpec((B, H), lambda i: (i, 0)),
          pl.BlockSpec((B, 1), lambda i: (i, 0)),
          pl.BlockSpec((1, H), lambda i: (0, 0)),
      ],
      out_specs=pl.BlockSpec((B, H), lambda i: (i, 0)),
      out_shape=jax.ShapeDtypeStruct((Np, H), jnp.float32),
  )(a1, hs, dinv_col, b1r)

  a2 = aggH_fn(h2s, src3, dst3, zH)

  out2, logp = pl.pallas_call(
      _k3_body,
      grid=grid,
      in_specs=[
          pl.BlockSpec((_NC, B, H), lambda i: (0, i, 0)),
          pl.BlockSpec((B, H), lambda i: (i, 0)),
          pl.BlockSpec((B, 1), lambda i: (i, 0)),
          pl.BlockSpec((H, C), lambda i: (0, 0)),
          pl.BlockSpec((1, C), lambda i: (0, 0)),
      ],
      out_specs=[
          pl.BlockSpec((B, C), lambda i: (i, 0)),
          pl.BlockSpec((B, C), lambda i: (i, 0)),
      ],
      out_shape=[
          jax.ShapeDtypeStruct((Np, C), jnp.float32),
          jax.ShapeDtypeStruct((Np, C), jnp.float32),
      ],
  )(a2, h2s, dinv_col, W2, b2r)

  return (out2[:N], logp[:N])


# speedup vs baseline: 30.8159x; 1.0834x over previous
import functools

import jax
import jax.numpy as jnp
from jax import lax
from jax.experimental import pallas as pl
from jax.experimental.pallas import tpu as pltpu
from jax.experimental.pallas import tpu_sc as plsc

_NC = 2
_NS = 16
_NW = _NC * _NS
_W = 128


def _make_edge_agg(Np, D, rows_per_worker, rows_per_tile):
  mesh = plsc.VectorSubcoreMesh(core_axis_name="c", subcore_axis_name="s")

  _IC = 40
  chunks = []
  base = 0
  while base < rows_per_worker:
    n = min(_IC, rows_per_worker - base)
    chunks.append((base, n))
    base += n

  @functools.partial(
      pl.kernel,
      out_type=jax.ShapeDtypeStruct((_NC, Np, D), jnp.float32),
      mesh=mesh,
      scratch_types=[
          pltpu.VMEM((_IC, _W), jnp.int32),
          pltpu.VMEM((_IC, _W), jnp.int32),
          pltpu.VMEM((_W, D), jnp.float32),
          pltpu.VMEM((_W, D), jnp.float32),
          pltpu.VMEM_SHARED((Np, D), jnp.float32),
          pltpu.SemaphoreType.DMA,
          pltpu.SemaphoreType.DMA,
      ],
  )
  def agg(rows_hbm, srcb, dstb, zeros_hbm, out,
          src_v, dst_v, buf0, buf1, accum, g0, g1):
    c = lax.axis_index("c")
    s = lax.axis_index("s")
    wid = c * _NS + s
    pltpu.sync_copy(zeros_hbm.at[pl.ds(s * rows_per_tile, rows_per_tile)],
                    accum.at[pl.ds(s * rows_per_tile, rows_per_tile)])
    plsc.subcore_barrier()

    for base, nw in chunks:
      pltpu.sync_copy(srcb.at[wid, pl.ds(base, nw)], src_v.at[pl.ds(0, nw)])
      pltpu.sync_copy(dstb.at[wid, pl.ds(base, nw)], dst_v.at[pl.ds(0, nw)])
      npair = (nw - 1) // 2
      pltpu.async_copy(rows_hbm.at[src_v.at[0]], buf0, g0)

      def pair(t, carry):
        j0 = 2 * t
        pltpu.async_copy(rows_hbm.at[src_v.at[j0 + 1]], buf1, g1)
        pltpu.make_async_copy(rows_hbm.at[src_v.at[j0]], buf0, g0).wait()
        pltpu.sync_copy(buf0, accum.at[dst_v.at[j0]], add=True)
        pltpu.async_copy(rows_hbm.at[src_v.at[j0 + 2]], buf0, g0)
        pltpu.make_async_copy(rows_hbm.at[src_v.at[j0 + 1]], buf1, g1).wait()
        pltpu.sync_copy(buf1, accum.at[dst_v.at[j0 + 1]], add=True)
        return carry

      lax.fori_loop(0, npair, pair, 0)
      j = 2 * npair
      pltpu.make_async_copy(rows_hbm.at[src_v.at[j]], buf0, g0).wait()
      if nw - j == 2:
        pltpu.async_copy(rows_hbm.at[src_v.at[j + 1]], buf1, g1)
        pltpu.sync_copy(buf0, accum.at[dst_v.at[j]], add=True)
        pltpu.make_async_copy(rows_hbm.at[src_v.at[j + 1]], buf1, g1).wait()
        pltpu.sync_copy(buf1, accum.at[dst_v.at[j + 1]], add=True)
      else:
        pltpu.sync_copy(buf0, accum.at[dst_v.at[j]], add=True)

    plsc.subcore_barrier()
    pltpu.sync_copy(accum.at[pl.ds(s * rows_per_tile, rows_per_tile)],
                    out.at[c, pl.ds(s * rows_per_tile, rows_per_tile)])

  return agg


def _make_deg(Np, rows_per_worker, rows_per_tile):
  mesh = plsc.VectorSubcoreMesh(core_axis_name="c", subcore_axis_name="s")

  @functools.partial(
      pl.kernel,
      out_type=jax.ShapeDtypeStruct((_NC * Np,), jnp.float32),
      mesh=mesh,
      scratch_types=[
          pltpu.VMEM((rows_per_worker, _W), jnp.int32),
          pltpu.VMEM((_W,), jnp.float32),
          pltpu.VMEM((rows_per_tile,), jnp.float32),
          pltpu.VMEM_SHARED((Np,), jnp.float32),
      ],
  )
  def deg(dstb, out, dst_v, ones_v, zbuf, accum):
    c = lax.axis_index("c")
    s = lax.axis_index("s")
    wid = c * _NS + s
    ones16 = jnp.ones((16,), jnp.float32)
    zero16 = jnp.zeros((16,), jnp.float32)

    def fill_ones(i, carry):
      ones_v[pl.ds(i * 16, 16)] = ones16
      return carry

    lax.fori_loop(0, _W // 16, fill_ones, 0)

    def fill_zero(i, carry):
      zbuf[pl.ds(i * 16, 16)] = zero16
      return carry

    lax.fori_loop(0, rows_per_tile // 16, fill_zero, 0)
    pltpu.sync_copy(zbuf, accum.at[pl.ds(s * rows_per_tile, rows_per_tile)])
    pltpu.sync_copy(dstb.at[wid], dst_v)
    plsc.subcore_barrier()

    def win(j, carry):
      pltpu.sync_copy(ones_v, accum.at[dst_v.at[j]], add=True)
      return carry

    lax.fori_loop(0, rows_per_worker, win, 0)
    plsc.subcore_barrier()
    obase = pl.multiple_of(c * Np + s * rows_per_tile, 128)
    pltpu.sync_copy(accum.at[pl.ds(s * rows_per_tile, rows_per_tile)],
                    out.at[pl.ds(obase, rows_per_tile)])

  return deg


def _k1_body(x_ref, w1_ref, dinv_ref, hs_ref):
  h = jnp.dot(x_ref[...], w1_ref[...], preferred_element_type=jnp.float32)
  hs_ref[...] = h * dinv_ref[...]


def _k2_body(a1_ref, hs_ref, dinv_ref, b1_ref, h2s_ref):
  dinv = dinv_ref[...]
  raw = a1_ref[0] + a1_ref[1] + hs_ref[...]
  h2 = jnp.maximum(raw * dinv + b1_ref[...], 0.0)
  h2s_ref[...] = h2 * dinv


def _k3_body(a2_ref, h2s_ref, dinv_ref, w2_ref, b2_ref, out_ref, logp_ref):
  raw = (a2_ref[0] + a2_ref[1] + h2s_ref[...]) * dinv_ref[...]
  o40 = jnp.dot(raw, w2_ref[...],
                preferred_element_type=jnp.float32) + b2_ref[...]
  m = jnp.max(o40, axis=1, keepdims=True)
  lse = jnp.log(jnp.sum(jnp.exp(o40 - m), axis=1, keepdims=True))
  out_ref[...] = o40
  logp_ref[...] = o40 - m - lse


def kernel(x, edge_index, W1, b1, W2, b2):
  N, F = x.shape
  H = W1.shape[1]
  C = W2.shape[1]
  E = edge_index.shape[1]
  Np = ((N + _NS * 128 - 1) // (_NS * 128)) * (_NS * 128)

  Ep = ((E + _W * _NW - 1) // (_W * _NW)) * (_W * _NW)
  n_rows = Ep // _W
  rows_per_worker = n_rows // _NW
  rows_per_tile = Np // _NS

  pad_idx = N + (jnp.arange(Ep - E, dtype=jnp.int32) % (Np - N))
  srcp = jnp.concatenate([edge_index[0], pad_idx])
  dstp = jnp.concatenate([edge_index[1], pad_idx])

  xp = jnp.pad(x, ((0, Np - N), (0, 0)))
  src3 = srcp.reshape(_NW, rows_per_worker, _W)
  dst3 = dstp.reshape(_NW, rows_per_worker, _W)
  zH = jnp.zeros((Np, H), jnp.float32)
  b1r = b1.reshape(1, H)
  b2r = b2.reshape(1, C)

  deg_fn = _make_deg(Np, rows_per_worker, rows_per_tile)
  aggH_fn = _make_edge_agg(Np, H, rows_per_worker, rows_per_tile)

  degp = deg_fn(dst3)
  dinv_col = lax.rsqrt(degp[:Np] + degp[Np:] + 1.0)[:, None]

  B = 1024
  grid = (Np // B,)
  hs = pl.pallas_call(
      _k1_body,
      grid=grid,
      in_specs=[
          pl.BlockSpec((B, F), lambda i: (i, 0)),
          pl.BlockSpec((F, H), lambda i: (0, 0)),
          pl.BlockSpec((B, 1), lambda i: (i, 0)),
      ],
      out_specs=pl.BlockSpec((B, H), lambda i: (i, 0)),
      out_shape=jax.ShapeDtypeStruct((Np, H), jnp.float32),
  )(xp, W1, dinv_col)

  a1 = aggH_fn(hs, src3, dst3, zH)

  h2s = pl.pallas_call(
      _k2_body,
      grid=grid,
      in_specs=[
          pl.BlockSpec((_NC, B, H), lambda i: (0, i, 0)),
          pl.BlockSpec((B, H), lambda i: (i, 0)),
          pl.BlockSpec((B, 1), lambda i: (i, 0)),
          pl.BlockSpec((1, H), lambda i: (0, 0)),
      ],
      out_specs=pl.BlockSpec((B, H), lambda i: (i, 0)),
      out_shape=jax.ShapeDtypeStruct((Np, H), jnp.float32),
  )(a1, hs, dinv_col, b1r)

  a2 = aggH_fn(h2s, src3, dst3, zH)

  out2, logp = pl.pallas_call(
      _k3_body,
      grid=grid,
      in_specs=[
          pl.BlockSpec((_NC, B, H), lambda i: (0, i, 0)),
          pl.BlockSpec((B, H), lambda i: (i, 0)),
          pl.BlockSpec((B, 1), lambda i: (i, 0)),
          pl.BlockSpec((H, C), lambda i: (0, 0)),
          pl.BlockSpec((1, C), lambda i: (0, 0)),
      ],
      out_specs=[
          pl.BlockSpec((B, C), lambda i: (i, 0)),
          pl.BlockSpec((B, C), lambda i: (i, 0)),
      ],
      out_shape=[
          jax.ShapeDtypeStruct((Np, C), jnp.float32),
          jax.ShapeDtypeStruct((Np, C), jnp.float32),
      ],
  )(a2, h2s, dinv_col, W2, b2r)

  return (out2[:N], logp[:N])
